# baseline (speedup 1.0000x reference)
import jax
import jax.numpy as jnp
from jax.experimental import pallas as pl
from jax.experimental.pallas import tpu as pltpu
from jax.sharding import PartitionSpec as P

_BLOCK = 5000
_NEG = -jnp.inf


def _mm(a, w):
    return jax.lax.dot_general(a, w, (((1,), (0,)), ((), ())),
                               preferred_element_type=jnp.float32)


def _bulk_kernel(x_ref,
                 sWr1, sb1, sWr2, sb2, sWr3,
                 cWr1, cb1, cWr2, cb2, cWr3,
                 smax_out, cmax_out,
                 smax, cmax):
    i = pl.program_id(0)
    nsteps = pl.num_programs(0)
    xb = x_ref[...]

    def gmax(h):
        return jnp.max(h.reshape(_BLOCK // 8, 8, 1024), axis=0)

    hs = jnp.maximum(_mm(xb, sWr1[...]) + sb1[...], 0.0)
    hs = jnp.maximum(_mm(hs, sWr2[...]) + sb2[...], 0.0)
    bs = gmax(_mm(hs, sWr3[...]))
    hc = jnp.maximum(_mm(xb, cWr1[...]) + cb1[...], 0.0)
    hc = jnp.maximum(_mm(hc, cWr2[...]) + cb2[...], 0.0)
    bc = gmax(_mm(hc, cWr3[...]))

    @pl.when(i == 0)
    def _init():
        smax[...] = jnp.full((8, 1024), _NEG, jnp.float32)
        cmax[...] = jnp.full((8, 1024), _NEG, jnp.float32)

    smax[...] = jnp.maximum(smax[...], bs)
    cmax[...] = jnp.maximum(cmax[...], bc)

    @pl.when(i == nsteps - 1)
    def _out():
        smax_out[...] = smax[...]
        cmax_out[...] = cmax[...]


def _tail_kernel(x8_ref,
                 sWr1, sWn1, sb1, sWr2, sWn2, sb2, sWr3, sWn3, sb3,
                 fc1W, fc1b, fc2W, fc2b, fc3W, fc3b,
                 cWr1, cWn1, cb1, cWr2, cWn2, cb2, cWr3, cWn3, cb3,
                 smax_ref, cmax_ref,
                 h_out, t9_out):
    x8 = x8_ref[...]
    rows = jax.lax.broadcasted_iota(jnp.int32, (8, 1), 0)
    sel = (rows == 1).astype(jnp.float32)
    keep = rows < 2

    def gconv(h, wr, wn):
        return _mm(h, wr[...]) + sel * _mm(h[0:1, :], wn[...])

    e = jnp.maximum(gconv(x8, sWr1, sWn1) + sb1[...], 0.0)
    e = jnp.maximum(gconv(e, sWr2, sWn2) + sb2[...], 0.0)
    e = gconv(e, sWr3, sWn3)
    es = jnp.max(jnp.where(keep, e, _NEG), axis=0, keepdims=True)
    e = jnp.maximum(gconv(x8, cWr1, cWn1) + cb1[...], 0.0)
    e = jnp.maximum(gconv(e, cWr2, cWn2) + cb2[...], 0.0)
    e = gconv(e, cWr3, cWn3)
    ec = jnp.max(jnp.where(keep, e, _NEG), axis=0, keepdims=True)

    bulk_s = jnp.max(smax_ref[...], axis=0, keepdims=True)
    bulk_c = jnp.max(cmax_ref[...], axis=0, keepdims=True)
    h_out[...] = jnp.maximum(bulk_c, ec) + cb3[...]
    s = jnp.maximum(jnp.maximum(bulk_s, es) + sb3[...], 0.0)
    t = jnp.maximum(_mm(s, fc1W[...]) + fc1b[...], 0.0)
    t = jnp.maximum(_mm(t, fc2W[...]) + fc2b[...], 0.0)
    t9 = _mm(t, fc3W[...]) + fc3b[...]
    col = jax.lax.broadcasted_iota(jnp.int32, (1, 9), 1)
    t9_out[...] = t9 + (col % 4 == 0).astype(jnp.float32)


def kernel(x, stn_g1_Wr, stn_g1_Wn, stn_g1_b, stn_g2_Wr, stn_g2_Wn, stn_g2_b,
           stn_g3_Wr, stn_g3_Wn, stn_g3_b, stn_fc1_W, stn_fc1_b,
           stn_fc2_W, stn_fc2_b, stn_fc3_W, stn_fc3_b,
           c1_Wr, c1_Wn, c1_b, c2_Wr, c2_Wn, c2_b, c3_Wr, c3_Wn, c3_b):
    n = x.shape[0]
    ndev = 2 if jax.device_count() >= 2 else 1

    x8 = x[0:8]
    x_bulk = x.at[1].set(x[0])

    row = lambda v: v.reshape(1, -1)
    bulk_w = (
        stn_g1_Wr, row(stn_g1_b), stn_g2_Wr, row(stn_g2_b), stn_g3_Wr,
        c1_Wr, row(c1_b), c2_Wr, row(c2_b), c3_Wr,
    )
    bspecs = [pl.BlockSpec(w.shape, lambda i: (0, 0)) for w in bulk_w]

    grid = n // ndev // _BLOCK
    assert grid * _BLOCK * ndev == n

    def bulk_fn(xs, *w):
        return pl.pallas_call(
            _bulk_kernel,
            grid=(grid,),
            in_specs=[pl.BlockSpec((_BLOCK, 3), lambda i: (i, 0))] + bspecs,
            out_specs=[pl.BlockSpec((8, 1024), lambda i: (0, 0)),
                       pl.BlockSpec((8, 1024), lambda i: (0, 0))],
            out_shape=[jax.ShapeDtypeStruct((8, 1024), jnp.float32),
                       jax.ShapeDtypeStruct((8, 1024), jnp.float32)],
            scratch_shapes=[pltpu.VMEM((8, 1024), jnp.float32),
                            pltpu.VMEM((8, 1024), jnp.float32)],
        )(xs, *w)

    def tail_fn(*args):
        return pl.pallas_call(
            _tail_kernel,
            out_shape=[jax.ShapeDtypeStruct((1, 1024), jnp.float32),
                       jax.ShapeDtypeStruct((1, 9), jnp.float32)],
        )(*args)

    def tail_args(smax, cmax):
        return (
            x8,
            stn_g1_Wr, stn_g1_Wn, row(stn_g1_b),
            stn_g2_Wr, stn_g2_Wn, row(stn_g2_b),
            stn_g3_Wr, stn_g3_Wn, row(stn_g3_b),
            stn_fc1_W, row(stn_fc1_b), stn_fc2_W, row(stn_fc2_b),
            stn_fc3_W, row(stn_fc3_b),
            c1_Wr, c1_Wn, row(c1_b),
            c2_Wr, c2_Wn, row(c2_b),
            c3_Wr, c3_Wn, row(c3_b),
            smax, cmax,
        )

    if ndev > 1:
        mesh = jax.make_mesh((ndev,), ("d",),
                             axis_types=(jax.sharding.AxisType.Explicit,))
        with jax.sharding.use_abstract_mesh(mesh.abstract_mesh):
            ns = lambda spec: jax.sharding.NamedSharding(mesh, spec)
            rep = lambda v: jax.reshard(v, ns(P(*(None,) * v.ndim)))
            x_sh = jax.reshard(x_bulk, ns(P("d", None)))
            w_sh = tuple(rep(w) for w in bulk_w)
            wspec = tuple(P(None, None) for _ in bulk_w)
            smax, cmax = jax.shard_map(
                bulk_fn, mesh=mesh,
                in_specs=(P("d", None),) + wspec,
                out_specs=(P("d", None), P("d", None)),
                check_vma=False,
            )(x_sh, *w_sh)
            targs = tuple(rep(a) for a in tail_args(smax, cmax))
            h, t9 = jax.shard_map(
                tail_fn, mesh=mesh,
                in_specs=tuple(P(*(None,) * a.ndim) for a in targs),
                out_specs=(P(None, None), P(None, None)),
                check_vma=False,
            )(*targs)
    else:
        smax, cmax = bulk_fn(x_bulk, *bulk_w)
        h, t9 = tail_fn(*tail_args(smax, cmax))
    return h, t9.reshape(3, 3)


# speedup vs baseline: 3.1631x; 3.1631x over previous
import jax
import jax.numpy as jnp
from jax.experimental import pallas as pl
from jax.experimental.pallas import tpu as pltpu

_BLOCK = 4000
_NEG = -jnp.inf


def _mm(a, w):
    return jax.lax.dot_general(a, w, (((1,), (0,)), ((), ())),
                               preferred_element_type=jnp.float32)


def _bulk_kernel(x_ref,
                 sWr1, sb1, sWr2, sb2, sWr3,
                 cWr1, cb1, cWr2, cb2, cWr3,
                 smax_out, cmax_out,
                 smax, cmax):
    i = pl.program_id(0)
    nsteps = pl.num_programs(0)
    xb = x_ref[...]

    def gmax(h):
        return jnp.max(h.reshape(_BLOCK // 8, 8, 1024), axis=0)

    hs = jnp.maximum(_mm(xb, sWr1[...]) + sb1[...], 0.0)
    hs = jnp.maximum(_mm(hs, sWr2[...]) + sb2[...], 0.0)
    bs = gmax(_mm(hs, sWr3[...]))
    hc = jnp.maximum(_mm(xb, cWr1[...]) + cb1[...], 0.0)
    hc = jnp.maximum(_mm(hc, cWr2[...]) + cb2[...], 0.0)
    bc = gmax(_mm(hc, cWr3[...]))

    @pl.when(i == 0)
    def _init():
        smax[...] = jnp.full((8, 1024), _NEG, jnp.float32)
        cmax[...] = jnp.full((8, 1024), _NEG, jnp.float32)

    smax[...] = jnp.maximum(smax[...], bs)
    cmax[...] = jnp.maximum(cmax[...], bc)

    @pl.when(i == nsteps - 1)
    def _out():
        smax_out[...] = smax[...]
        cmax_out[...] = cmax[...]


def _tail_kernel(x8_ref,
                 sWr1, sWn1, sb1, sWr2, sWn2, sb2, sWr3, sWn3, sb3,
                 fc1W, fc1b, fc2W, fc2b, fc3W, fc3b,
                 cWr1, cWn1, cb1, cWr2, cWn2, cb2, cWr3, cWn3, cb3,
                 smax_ref, cmax_ref,
                 h_out, t9_out):
    x8 = x8_ref[...]
    rows = jax.lax.broadcasted_iota(jnp.int32, (8, 1), 0)
    sel = (rows == 1).astype(jnp.float32)
    keep = rows < 2

    def gconv(h, wr, wn):
        return _mm(h, wr[...]) + sel * _mm(h[0:1, :], wn[...])

    e = jnp.maximum(gconv(x8, sWr1, sWn1) + sb1[...], 0.0)
    e = jnp.maximum(gconv(e, sWr2, sWn2) + sb2[...], 0.0)
    e = gconv(e, sWr3, sWn3)
    es = jnp.max(jnp.where(keep, e, _NEG), axis=0, keepdims=True)
    e = jnp.maximum(gconv(x8, cWr1, cWn1) + cb1[...], 0.0)
    e = jnp.maximum(gconv(e, cWr2, cWn2) + cb2[...], 0.0)
    e = gconv(e, cWr3, cWn3)
    ec = jnp.max(jnp.where(keep, e, _NEG), axis=0, keepdims=True)

    bulk_s = jnp.max(smax_ref[...], axis=0, keepdims=True)
    bulk_c = jnp.max(cmax_ref[...], axis=0, keepdims=True)
    h_out[...] = jnp.maximum(bulk_c, ec) + cb3[...]
    s = jnp.maximum(jnp.maximum(bulk_s, es) + sb3[...], 0.0)
    t = jnp.maximum(_mm(s, fc1W[...]) + fc1b[...], 0.0)
    t = jnp.maximum(_mm(t, fc2W[...]) + fc2b[...], 0.0)
    t9 = _mm(t, fc3W[...]) + fc3b[...]
    col = jax.lax.broadcasted_iota(jnp.int32, (1, 9), 1)
    t9_out[...] = t9 + (col % 4 == 0).astype(jnp.float32)


def kernel(x, stn_g1_Wr, stn_g1_Wn, stn_g1_b, stn_g2_Wr, stn_g2_Wn, stn_g2_b,
           stn_g3_Wr, stn_g3_Wn, stn_g3_b, stn_fc1_W, stn_fc1_b,
           stn_fc2_W, stn_fc2_b, stn_fc3_W, stn_fc3_b,
           c1_Wr, c1_Wn, c1_b, c2_Wr, c2_Wn, c2_b, c3_Wr, c3_Wn, c3_b):
    n = x.shape[0]
    ndev = 1

    x8 = x[0:8]
    x_bulk = x.at[1].set(x[0])

    row = lambda v: v.reshape(1, -1)
    bulk_w = (
        stn_g1_Wr, row(stn_g1_b), stn_g2_Wr, row(stn_g2_b), stn_g3_Wr,
        c1_Wr, row(c1_b), c2_Wr, row(c2_b), c3_Wr,
    )
    bspecs = [pl.BlockSpec(w.shape, lambda i: (0, 0)) for w in bulk_w]

    grid = n // ndev // _BLOCK
    assert grid * _BLOCK * ndev == n

    def bulk_fn(xs, *w):
        return pl.pallas_call(
            _bulk_kernel,
            grid=(grid,),
            in_specs=[pl.BlockSpec((_BLOCK, 3), lambda i: (i, 0))] + bspecs,
            out_specs=[pl.BlockSpec((8, 1024), lambda i: (0, 0)),
                       pl.BlockSpec((8, 1024), lambda i: (0, 0))],
            out_shape=[jax.ShapeDtypeStruct((8, 1024), jnp.float32),
                       jax.ShapeDtypeStruct((8, 1024), jnp.float32)],
            scratch_shapes=[pltpu.VMEM((8, 1024), jnp.float32),
                            pltpu.VMEM((8, 1024), jnp.float32)],
        )(xs, *w)

    def tail_fn(*args):
        return pl.pallas_call(
            _tail_kernel,
            out_shape=[jax.ShapeDtypeStruct((1, 1024), jnp.float32),
                       jax.ShapeDtypeStruct((1, 9), jnp.float32)],
        )(*args)

    def tail_args(smax, cmax):
        return (
            x8,
            stn_g1_Wr, stn_g1_Wn, row(stn_g1_b),
            stn_g2_Wr, stn_g2_Wn, row(stn_g2_b),
            stn_g3_Wr, stn_g3_Wn, row(stn_g3_b),
            stn_fc1_W, row(stn_fc1_b), stn_fc2_W, row(stn_fc2_b),
            stn_fc3_W, row(stn_fc3_b),
            c1_Wr, c1_Wn, row(c1_b),
            c2_Wr, c2_Wn, row(c2_b),
            c3_Wr, c3_Wn, row(c3_b),
            smax, cmax,
        )

    smax, cmax = bulk_fn(x_bulk, *bulk_w)
    h, t9 = tail_fn(*tail_args(smax, cmax))
    return h, t9.reshape(3, 3)


# speedup vs baseline: 3.2325x; 1.0219x over previous
import jax
import jax.numpy as jnp
from jax.experimental import pallas as pl
from jax.experimental.pallas import tpu as pltpu

_BLOCK = 10000
_NEG = -jnp.inf


def _mm(a, w):
    return jax.lax.dot_general(a, w, (((1,), (0,)), ((), ())),
                               preferred_element_type=jnp.float32)


def _bulk_kernel(x_ref,
                 sWr1, sb1, sWr2, sb2, sWr3,
                 cWr1, cb1, cWr2, cb2, cWr3,
                 smax_out, cmax_out,
                 smax, cmax):
    i = pl.program_id(0)
    nsteps = pl.num_programs(0)
    xb = x_ref[...]

    def gmax(h):
        return jnp.max(h.reshape(_BLOCK // 8, 8, 1024), axis=0)

    hs = jnp.maximum(_mm(xb, sWr1[...]) + sb1[...], 0.0)
    hs = jnp.maximum(_mm(hs, sWr2[...]) + sb2[...], 0.0)
    bs = gmax(_mm(hs, sWr3[...]))
    hc = jnp.maximum(_mm(xb, cWr1[...]) + cb1[...], 0.0)
    hc = jnp.maximum(_mm(hc, cWr2[...]) + cb2[...], 0.0)
    bc = gmax(_mm(hc, cWr3[...]))

    @pl.when(i == 0)
    def _init():
        smax[...] = jnp.full((8, 1024), _NEG, jnp.float32)
        cmax[...] = jnp.full((8, 1024), _NEG, jnp.float32)

    smax[...] = jnp.maximum(smax[...], bs)
    cmax[...] = jnp.maximum(cmax[...], bc)

    @pl.when(i == nsteps - 1)
    def _out():
        smax_out[...] = smax[...]
        cmax_out[...] = cmax[...]


def _tail_kernel(x8_ref,
                 sWr1, sWn1, sb1, sWr2, sWn2, sb2, sWr3, sWn3, sb3,
                 fc1W, fc1b, fc2W, fc2b, fc3W, fc3b,
                 cWr1, cWn1, cb1, cWr2, cWn2, cb2, cWr3, cWn3, cb3,
                 smax_ref, cmax_ref,
                 h_out, t9_out):
    x8 = x8_ref[...]
    rows = jax.lax.broadcasted_iota(jnp.int32, (8, 1), 0)
    sel = (rows == 1).astype(jnp.float32)
    keep = rows < 2

    def gconv(h, wr, wn):
        return _mm(h, wr[...]) + sel * _mm(h[0:1, :], wn[...])

    e = jnp.maximum(gconv(x8, sWr1, sWn1) + sb1[...], 0.0)
    e = jnp.maximum(gconv(e, sWr2, sWn2) + sb2[...], 0.0)
    e = gconv(e, sWr3, sWn3)
    es = jnp.max(jnp.where(keep, e, _NEG), axis=0, keepdims=True)
    e = jnp.maximum(gconv(x8, cWr1, cWn1) + cb1[...], 0.0)
    e = jnp.maximum(gconv(e, cWr2, cWn2) + cb2[...], 0.0)
    e = gconv(e, cWr3, cWn3)
    ec = jnp.max(jnp.where(keep, e, _NEG), axis=0, keepdims=True)

    bulk_s = jnp.max(smax_ref[...], axis=0, keepdims=True)
    bulk_c = jnp.max(cmax_ref[...], axis=0, keepdims=True)
    h_out[...] = jnp.maximum(bulk_c, ec) + cb3[...]
    s = jnp.maximum(jnp.maximum(bulk_s, es) + sb3[...], 0.0)
    t = jnp.maximum(_mm(s, fc1W[...]) + fc1b[...], 0.0)
    t = jnp.maximum(_mm(t, fc2W[...]) + fc2b[...], 0.0)
    t9 = _mm(t, fc3W[...]) + fc3b[...]
    col = jax.lax.broadcasted_iota(jnp.int32, (1, 9), 1)
    t9_out[...] = t9 + (col % 4 == 0).astype(jnp.float32)


def kernel(x, stn_g1_Wr, stn_g1_Wn, stn_g1_b, stn_g2_Wr, stn_g2_Wn, stn_g2_b,
           stn_g3_Wr, stn_g3_Wn, stn_g3_b, stn_fc1_W, stn_fc1_b,
           stn_fc2_W, stn_fc2_b, stn_fc3_W, stn_fc3_b,
           c1_Wr, c1_Wn, c1_b, c2_Wr, c2_Wn, c2_b, c3_Wr, c3_Wn, c3_b):
    n = x.shape[0]
    ndev = 1

    x8 = x[0:8]
    x_bulk = x.at[1].set(x[0])

    row = lambda v: v.reshape(1, -1)
    bulk_w = (
        stn_g1_Wr, row(stn_g1_b), stn_g2_Wr, row(stn_g2_b), stn_g3_Wr,
        c1_Wr, row(c1_b), c2_Wr, row(c2_b), c3_Wr,
    )
    bspecs = [pl.BlockSpec(w.shape, lambda i: (0, 0)) for w in bulk_w]

    grid = n // ndev // _BLOCK
    assert grid * _BLOCK * ndev == n

    def bulk_fn(xs, *w):
        return pl.pallas_call(
            _bulk_kernel,
            grid=(grid,),
            in_specs=[pl.BlockSpec((_BLOCK, 3), lambda i: (i, 0))] + bspecs,
            out_specs=[pl.BlockSpec((8, 1024), lambda i: (0, 0)),
                       pl.BlockSpec((8, 1024), lambda i: (0, 0))],
            out_shape=[jax.ShapeDtypeStruct((8, 1024), jnp.float32),
                       jax.ShapeDtypeStruct((8, 1024), jnp.float32)],
            scratch_shapes=[pltpu.VMEM((8, 1024), jnp.float32),
                            pltpu.VMEM((8, 1024), jnp.float32)],
        )(xs, *w)

    def tail_fn(*args):
        return pl.pallas_call(
            _tail_kernel,
            out_shape=[jax.ShapeDtypeStruct((1, 1024), jnp.float32),
                       jax.ShapeDtypeStruct((1, 9), jnp.float32)],
        )(*args)

    def tail_args(smax, cmax):
        return (
            x8,
            stn_g1_Wr, stn_g1_Wn, row(stn_g1_b),
            stn_g2_Wr, stn_g2_Wn, row(stn_g2_b),
            stn_g3_Wr, stn_g3_Wn, row(stn_g3_b),
            stn_fc1_W, row(stn_fc1_b), stn_fc2_W, row(stn_fc2_b),
            stn_fc3_W, row(stn_fc3_b),
            c1_Wr, c1_Wn, row(c1_b),
            c2_Wr, c2_Wn, row(c2_b),
            c3_Wr, c3_Wn, row(c3_b),
            smax, cmax,
        )

    smax, cmax = bulk_fn(x_bulk, *bulk_w)
    h, t9 = tail_fn(*tail_args(smax, cmax))
    return h, t9.reshape(3, 3)
